# baseline (speedup 1.0000x reference)
import functools

import jax
import jax.numpy as jnp
from jax import lax
from jax.experimental import pallas as pl
from jax.experimental.pallas import tpu as pltpu
from jax.experimental.pallas import tpu_sc as plsc

EMBED_DIM = 64
NUM_CLS = 16
SEQ = 200
CW = 128
TAIL = SEQ - CW
LANES = 16


def _tc_project(emb_t, w):
  v = emb_t.shape[1]
  bn = 6400
  grid = (v + bn - 1) // bn

  def body(x_ref, w_ref, o_ref):
    ws = w_ref[...] * (1.0 / SEQ)
    o_ref[...] = lax.dot_general(
        ws, x_ref[...], (((1,), (0,)), ((), ())),
        preferred_element_type=jnp.float32,
    )

  return pl.pallas_call(
      body,
      grid=(grid,),
      in_specs=[
          pl.BlockSpec((EMBED_DIM, bn), lambda g: (0, g)),
          pl.BlockSpec((NUM_CLS, EMBED_DIM), lambda g: (0, 0)),
      ],
      out_specs=pl.BlockSpec((NUM_CLS, bn), lambda g: (0, g)),
      out_shape=jax.ShapeDtypeStruct((NUM_CLS, v), jnp.float32),
  )(emb_t, w)


@functools.cache
def _sc_gather(batch, vocab):
  info = plsc.get_sparse_core_info()
  num_workers = info.num_cores * info.num_subcores
  bpw = batch // num_workers
  nbuf = 8
  mesh = plsc.VectorSubcoreMesh(core_axis_name="c", subcore_axis_name="s")

  @functools.partial(
      pl.kernel,
      out_type=jax.ShapeDtypeStruct((batch, NUM_CLS), jnp.float32),
      mesh=mesh,
      scratch_types=[
          pltpu.VMEM((bpw, CW), jnp.int32),
          pltpu.VMEM((bpw, CW), jnp.int32),
          pltpu.VMEM((nbuf, CW, NUM_CLS), jnp.float32),
          pltpu.VMEM((bpw, NUM_CLS), jnp.float32),
          pltpu.VMEM((LANES,), jnp.float32),
          pltpu.SemaphoreType.DMA,
      ],
      compiler_params=pltpu.CompilerParams(use_tc_tiling_on_sc=False),
  )
  def sc_gather(
      ids_a_hbm, ids_b_hbm, tab_hbm, b_hbm, out_hbm, idx_a, idx_b, rows_v,
      out_v, b_v, sem,
  ):
    wid = lax.axis_index("s") * info.num_cores + lax.axis_index("c")
    pltpu.sync_copy(ids_a_hbm.at[pl.ds(wid * bpw, bpw)], idx_a)
    pltpu.sync_copy(ids_b_hbm.at[pl.ds(wid * bpw, bpw)], idx_b)
    pltpu.sync_copy(b_hbm, b_v)
    bias = b_v[...]

    def dma(row, parity, buf):
      if parity == 0:
        return pltpu.make_async_copy(
            tab_hbm.at[idx_a.at[row]], rows_v.at[buf], sem
        )
      return pltpu.make_async_copy(
          tab_hbm.at[idx_b.at[row, pl.ds(0, TAIL)]],
          rows_v.at[buf, pl.ds(0, TAIL)],
          sem,
      )

    for c in range(nbuf - 1):
      dma(c // 2, c % 2, c).start()

    def reduce_span(buf, hi, acc):
      def body(r, a):
        return a + rows_v[buf, r]

      return lax.fori_loop(0, hi, body, acc, unroll=8)

    def outer(g, carry):
      row0 = g * (nbuf // 2)
      for c in range(nbuf):
        parity = c % 2
        row = row0 + c // 2
        nxt_row = row0 + (c + nbuf - 1) // 2

        @pl.when(nxt_row < bpw)
        def _():
          dma(nxt_row, (c + nbuf - 1) % 2, (c + nbuf - 1) % nbuf).start()

        dma(row, parity, c).wait()
        if parity == 0:
          acc = reduce_span(c, CW, jnp.zeros((LANES,), jnp.float32))
        else:
          acc = reduce_span(c, TAIL, acc)
          out_v[row] = acc + bias
      return carry

    lax.fori_loop(0, 2 * bpw // nbuf, outer, 0)
    pltpu.sync_copy(out_v, out_hbm.at[pl.ds(wid * bpw, bpw)])

  return sc_gather


def kernel(text_ids, length, emb, W, b):
  del length
  batch = text_ids.shape[0]
  vocab = emb.shape[0]
  ids_a = text_ids[:, :CW]
  ids_b = jnp.pad(text_ids[:, CW:], ((0, 0), (0, 2 * CW - SEQ)))
  table = jnp.transpose(_tc_project(jnp.transpose(emb), W))
  return _sc_gather(batch, vocab)(ids_a, ids_b, table, b)


# speedup vs baseline: 1.1431x; 1.1431x over previous
import functools

import jax
import jax.numpy as jnp
from jax import lax
from jax.experimental import pallas as pl
from jax.experimental.pallas import tpu as pltpu
from jax.experimental.pallas import tpu_sc as plsc

EMBED_DIM = 64
NUM_CLS = 16
SEQ = 200
CHUNKS = ((0, 200),)
LANES = 16
NQ = EMBED_DIM // LANES


@functools.cache
def _sc_pool(batch):
  info = plsc.get_sparse_core_info()
  num_workers = info.num_cores * info.num_subcores
  bpw = batch // num_workers
  nbuf = 4
  mesh = plsc.VectorSubcoreMesh(core_axis_name="c", subcore_axis_name="s")

  @functools.partial(
      pl.kernel,
      out_type=jax.ShapeDtypeStruct((batch, EMBED_DIM), jnp.float32),
      mesh=mesh,
      scratch_types=[
          pltpu.VMEM((bpw, SEQ), jnp.int32),
          pltpu.VMEM((nbuf, SEQ, EMBED_DIM), jnp.float32),
          pltpu.VMEM((bpw, EMBED_DIM), jnp.float32),
          pltpu.SemaphoreType.DMA,
      ],
      compiler_params=pltpu.CompilerParams(use_tc_tiling_on_sc=False),
  )
  def sc_pool(ids_hbm, emb_hbm, out_hbm, idx_v, rows_v, pooled_v, sem):
    wid = lax.axis_index("s") * info.num_cores + lax.axis_index("c")
    base = wid * bpw
    pltpu.sync_copy(ids_hbm.at[pl.ds(base, bpw)], idx_v)

    def row_dmas(row, buf):
      return [
          pltpu.make_async_copy(
              emb_hbm.at[idx_v.at[row, pl.ds(off, sz)]],
              rows_v.at[buf, pl.ds(off, sz)],
              sem,
          )
          for off, sz in CHUNKS
      ]

    def fire(row, buf):
      for dma in row_dmas(row, buf):
        dma.start()

    def drain_reduce(row, buf):
      for dma in row_dmas(row, buf):
        dma.wait()
      zero = jnp.zeros((LANES,), jnp.float32)

      def body(r, acc):
        return tuple(
            acc[q] + rows_v[buf, r, pl.ds(q * LANES, LANES)]
            for q in range(NQ)
        )

      acc = lax.fori_loop(0, SEQ, body, (zero,) * NQ, unroll=4)
      for q in range(NQ):
        pooled_v[row, pl.ds(q * LANES, LANES)] = acc[q] * (1.0 / SEQ)

    for i in range(nbuf - 1):
      fire(i, i)

    def outer(g, carry):
      for b in range(nbuf):
        row = g * nbuf + b

        @pl.when(row + nbuf - 1 < bpw)
        def _():
          fire(row + nbuf - 1, (b + nbuf - 1) % nbuf)

        drain_reduce(row, b)
      return carry

    lax.fori_loop(0, bpw // nbuf, outer, 0)
    pltpu.sync_copy(pooled_v, out_hbm.at[pl.ds(base, bpw)])

  return sc_pool


def _tc_head(pooled, w_t, bias):
  def body(p_ref, w_ref, b_ref, o_ref):
    o_ref[...] = (
        jnp.dot(p_ref[...], w_ref[...], preferred_element_type=jnp.float32)
        + b_ref[...]
    )

  return pl.pallas_call(
      body,
      out_shape=jax.ShapeDtypeStruct((pooled.shape[0], NUM_CLS), jnp.float32),
  )(pooled, w_t, bias)


def kernel(text_ids, length, emb, W, b):
  del length
  pooled = _sc_pool(text_ids.shape[0])(text_ids, emb)
  return _tc_head(pooled, W.T, b.reshape(1, NUM_CLS))
